# baseline (speedup 1.0000x reference)
import functools

import jax
import jax.numpy as jnp
from jax.experimental import pallas as pl
from jax.experimental.pallas import tpu as pltpu
from jax.experimental.pallas import tpu_sc as plsc

EMB = 128

_MESH = plsc.ScalarSubcoreMesh(axis_name="c", num_cores=1)


@functools.partial(
    pl.kernel,
    mesh=_MESH,
    out_type=jax.ShapeDtypeStruct((EMB,), jnp.float32),
    scratch_types=[
        pltpu.SMEM((1,), jnp.int32),
    ],
    compiler_params=pltpu.CompilerParams(skip_device_barrier=True),
)
def _sc_lookup(idx_hbm, table_hbm, out_hbm, idx_s):
    pltpu.sync_copy(idx_hbm, idx_s)
    tok = idx_s[0]
    pltpu.sync_copy(table_hbm.at[tok], out_hbm)


def kernel(token, table):
    idx = jnp.reshape(jnp.asarray(token, jnp.int32), (1,))
    return _sc_lookup(idx, table)


# speedup vs baseline: 1.0302x; 1.0302x over previous
import functools

import jax
import jax.numpy as jnp
from jax.experimental import pallas as pl
from jax.experimental.pallas import tpu as pltpu
from jax.experimental.pallas import tpu_sc as plsc

EMB = 128

_MESH = plsc.ScalarSubcoreMesh(axis_name="c", num_cores=1)


@functools.partial(
    pl.kernel,
    mesh=_MESH,
    out_type=jax.ShapeDtypeStruct((EMB,), jnp.float32),
    scratch_types=[
        pltpu.SMEM((1,), jnp.int32),
    ],
)
def _sc_lookup(idx_hbm, table_hbm, out_hbm, idx_s):
    pltpu.sync_copy(idx_hbm, idx_s)
    tok = idx_s[0]
    pltpu.sync_copy(table_hbm.at[tok], out_hbm)


def kernel(token, table):
    idx = jnp.reshape(jnp.asarray(token, jnp.int32), (1,))
    return _sc_lookup(idx, table)
